# Initial kernel scaffold; baseline (speedup 1.0000x reference)
import jax
import jax.numpy as jnp
from jax.experimental import pallas as pl


def kernel(enc_hidden, query_hidden, Wg, bg, Wq, bq, Wk, bk, Wo, bo):
    raise NotImplementedError("write your pallas kernel here")


# speedup vs baseline: 4.5439x; 4.5439x over previous
import jax
import jax.numpy as jnp
from jax.experimental import pallas as pl
from jax.experimental.pallas import tpu as pltpu

HIDDEN_DIM = 256
MEMORY_SLOTS_FULL = 512
MEMORY_SLOTS = 128
VOCAB_SIZE = 64
EPS = 1e-8


def _dot_t(a, b):
    return jax.lax.dot_general(
        a, b, (((1,), (1,)), ((), ())), preferred_element_type=jnp.float32)


def _elt(x2, b):
    return jnp.sum(jax.lax.slice(x2, (b, 0), (b + 1, 1)))


def _fused_kernel(enc2d_ref, encT_ref, q_ref, Wg_ref, bg_ref, Wq_ref, bq_ref,
                  Wk_ref, bk_ref, Wo_ref, bo_ref,
                  logits_ref, gp_ref, wr_ref,
                  mem_ref):
    BT, H = enc2d_ref.shape
    T, B, _ = encT_ref.shape
    M = mem_ref.shape[1]
    f32 = jnp.float32

    wg = Wg_ref[...]
    bg = bg_ref[0, 0]
    gp_ref[...] = jax.nn.sigmoid(
        jnp.sum(enc2d_ref[...] * wg, axis=-1, keepdims=True) + bg)

    mem_ref[...] = jnp.zeros((B, M, H), f32)

    slot_iota = jax.lax.broadcasted_iota(jnp.int32, (B, M), 1)

    def step(t, carry):
        n2, inv, total_writes = carry
        tok = encT_ref[pl.ds(t, 1), :, :].reshape(B, H)
        gate = jnp.sum(tok * wg, axis=-1, keepdims=True) + bg
        d = jnp.concatenate(
            [_dot_t(tok[b:b + 1, :], mem_ref[b]) for b in range(B)],
            axis=0) * inv
        dmin = jnp.min(d, axis=-1, keepdims=True)
        best = jnp.min(jnp.where(d == dmin, slot_iota, M),
                       axis=-1, keepdims=True)
        onehot = slot_iota == best
        n2_old = jnp.sum(jnp.where(onehot, n2, 0.0),
                         axis=-1, keepdims=True)
        n2_tok = jnp.sum(tok * tok, axis=-1, keepdims=True)
        do_w = jnp.logical_and(n2_tok > n2_old, gate > 0.0)
        ohw = jnp.logical_and(onehot, do_w)
        n2 = jnp.where(ohw, n2_tok, n2)
        inv_tok = 1.0 / jnp.maximum(jnp.sqrt(n2_tok), EPS)
        inv = jnp.where(ohw, inv_tok, inv)
        for b in range(B):
            best_b = _elt(best, b)
            do_b = _elt(do_w.astype(jnp.int32), b) > 0
            @pl.when(do_b)
            def _():
                mem_ref[b, pl.ds(best_b, 1), :] = tok[b:b + 1, :]
        total_writes = total_writes + jnp.sum(do_w.astype(f32))
        return n2, inv, total_writes

    init = (jnp.zeros((B, M), f32), jnp.full((B, M), 1.0 / EPS, f32),
            jnp.float32(0.0))
    _, _, total_writes = jax.lax.fori_loop(0, T - 3, step, init)
    wr_ref[...] = (total_writes / (B * (T - 3))).reshape(1, 1)

    qh = q_ref[...]
    q = _dot_t(qh, Wq_ref[...]) + bq_ref[...]
    bk = bk_ref[...]
    u = jax.lax.dot_general(q, Wk_ref[...], (((1,), (0,)), ((), ())),
                            preferred_element_type=f32)
    scale = 1.0 / (H ** 0.5)
    s_zero = jnp.sum(q * bk, axis=-1, keepdims=True) * scale
    scores = jnp.concatenate(
        [_dot_t(u[b:b + 1, :], mem_ref[b]) for b in range(B)],
        axis=0) * scale + s_zero
    m = jnp.maximum(jnp.max(scores, axis=-1, keepdims=True), s_zero)
    e = jnp.exp(scores - m)
    denom = (jnp.sum(e, axis=-1, keepdims=True)
             + (MEMORY_SLOTS_FULL - M) * jnp.exp(s_zero - m))
    attn = e / denom
    retrieved = jnp.concatenate(
        [jax.lax.dot_general(attn[b:b + 1, :], mem_ref[b],
                             (((1,), (0,)), ((), ())),
                             preferred_element_type=f32)
         for b in range(B)], axis=0)
    logits_ref[...] = _dot_t(retrieved + qh, Wo_ref[...]) + bo_ref[...]


def kernel(enc_hidden, query_hidden, Wg, bg, Wq, bq, Wk, bk, Wo, bo):
    B, T, H = enc_hidden.shape
    f32 = jnp.float32
    out_shapes = (
        jax.ShapeDtypeStruct((B, VOCAB_SIZE), f32),
        jax.ShapeDtypeStruct((B * T, 1), f32),
        jax.ShapeDtypeStruct((1, 1), f32),
    )
    logits, gate_probs, wr = pl.pallas_call(
        _fused_kernel,
        out_shape=out_shapes,
        scratch_shapes=[
            pltpu.VMEM((B, MEMORY_SLOTS, H), f32),
        ],
    )(
        enc_hidden.reshape(B * T, H), jnp.swapaxes(enc_hidden, 0, 1),
        query_hidden, Wg,
        bg.reshape(1, 1), Wq, bq.reshape(1, H),
        Wk, bk.reshape(1, H), Wo, bo.reshape(1, VOCAB_SIZE),
    )
    return logits, gate_probs.reshape(B, T), wr[0, 0]


# speedup vs baseline: 13.4527x; 2.9606x over previous
import functools

import jax
import jax.numpy as jnp
from jax import lax
from jax.experimental import pallas as pl
from jax.experimental.pallas import tpu as pltpu
from jax.experimental.pallas import tpu_sc as plsc

HIDDEN_DIM = 256
MEMORY_SLOTS_FULL = 512
MEMORY_SLOTS = 128
VOCAB_SIZE = 64
EPS = 1e-8
LANES = 16


def _dot_t(a, b):
    return jax.lax.dot_general(
        a, b, (((1,), (1,)), ((), ())), preferred_element_type=jnp.float32)


def _pre_kernel(enc_ref, enc2d_ref, q_ref, Wg_ref, bg_ref, Wq_ref, bq_ref,
                Wk_ref, bk_ref,
                gp_ref, G_ref, n2f_ref, invnf_ref, wvalf_ref, kq_ref):
    B, T, H = enc_ref.shape
    f32 = jnp.float32
    wg = Wg_ref[...]
    bg = bg_ref[0, 0]
    enc2d = enc2d_ref[...]
    gs = jnp.sum(enc2d * wg, axis=-1, keepdims=True) + bg
    gp_ref[...] = jax.nn.sigmoid(gs)
    n2f = jnp.sum(enc2d * enc2d, axis=-1, keepdims=True)
    n2f_ref[...] = n2f
    invnf_ref[...] = 1.0 / jnp.maximum(jnp.sqrt(n2f), EPS)
    wvalf_ref[...] = jnp.where(gs > 0, n2f, -1.0)
    q = _dot_t(q_ref[...], Wq_ref[...]) + bq_ref[...]
    u = lax.dot_general(q, Wk_ref[...], (((1,), (0,)), ((), ())),
                        preferred_element_type=f32)
    scale = 1.0 / (H ** 0.5)
    for b in range(B):
        G_ref[b] = _dot_t(enc_ref[b], enc_ref[b])
    kq_ref[...] = jnp.concatenate(
        [_dot_t(u[b:b + 1, :], enc_ref[b]) for b in range(B)],
        axis=0) * scale


def _sc_scan(G, n2, invn, wval, kq):
    B, T = n2.shape
    M = MEMORY_SLOTS
    L = LANES
    NCH = M // L
    f32 = jnp.float32
    i32 = jnp.int32
    mesh = plsc.VectorSubcoreMesh(core_axis_name="c", subcore_axis_name="s")

    @functools.partial(
        pl.kernel,
        mesh=mesh,
        compiler_params=pltpu.CompilerParams(needs_layout_passes=False),
        out_type=[jax.ShapeDtypeStruct((B, T), f32),
                  jax.ShapeDtypeStruct((B, L), f32)],
        scratch_types=[
            pltpu.VMEM((T * T,), f32),
            pltpu.VMEM((T,), f32),
            pltpu.VMEM((T,), f32),
            pltpu.VMEM((T,), f32),
            pltpu.VMEM((T,), f32),
            pltpu.VMEM((M,), i32),
            pltpu.VMEM((M,), f32),
            pltpu.VMEM((M,), f32),
            pltpu.VMEM((M,), f32),
            pltpu.VMEM((T,), f32),
            pltpu.VMEM((L,), f32),
        ],
    )
    def scan_kernel(G_hbm, n2_hbm, invn_hbm, wval_hbm, kq_hbm,
                    attn_hbm, nw_hbm,
                    g_v, n2_v, invn_v, wval_v, kq_v,
                    src_v, sinv_v, sn2_v, d_v, attn_v, nw_v):
        wid = lax.axis_index("s") * 2 + lax.axis_index("c")

        @pl.when(wid < B)
        def _body():
            b = wid
            pltpu.sync_copy(G_hbm.at[b], g_v)
            pltpu.sync_copy(n2_hbm.at[b], n2_v)
            pltpu.sync_copy(invn_hbm.at[b], invn_v)
            pltpu.sync_copy(wval_hbm.at[b], wval_v)
            pltpu.sync_copy(kq_hbm.at[b], kq_v)
            zf = jnp.zeros((L,), f32)
            zi = jnp.zeros((L,), i32)
            for c in range(NCH):
                src_v[pl.ds(c * L, L)] = zi
                sinv_v[pl.ds(c * L, L)] = zf
                sn2_v[pl.ds(c * L, L)] = zf
            lane_iota = lax.broadcasted_iota(i32, (L,), 0)
            lane0 = lane_iota == 0
            BIG = jnp.int32(1 << 30)

            def step(t, writes):
                tbase = t * T
                minv = jnp.full((L,), jnp.inf, f32)
                for c in range(NCH):
                    idx = src_v[pl.ds(c * L, L)]
                    gv = plsc.load_gather(g_v, [idx + tbase])
                    d = gv * sinv_v[pl.ds(c * L, L)]
                    d_v[pl.ds(c * L, L)] = d
                    minv = jnp.minimum(minv, d)
                m = jnp.min(minv)
                bestv = jnp.full((L,), BIG, i32)
                for c in range(NCH):
                    d = d_v[pl.ds(c * L, L)]
                    bestv = jnp.minimum(
                        bestv, jnp.where(d == m, lane_iota + c * L, BIG))
                best = jnp.min(bestv)
                bestx = jnp.full((L,), best, i32)
                tx = jnp.full((L,), t, i32)
                n2old = plsc.load_gather(sn2_v, [bestx])
                wv = plsc.load_gather(wval_v, [tx])
                do = jnp.min((wv > n2old).astype(i32)) > 0

                @pl.when(do)
                def _write():
                    plsc.store_scatter(src_v, [bestx], tx, mask=lane0)
                    plsc.store_scatter(sinv_v, [bestx],
                                       plsc.load_gather(invn_v, [tx]),
                                       mask=lane0)
                    plsc.store_scatter(sn2_v, [bestx],
                                       plsc.load_gather(n2_v, [tx]),
                                       mask=lane0)

                return writes + jnp.where(do, 1.0, 0.0)

            writes = lax.fori_loop(0, T - 3, step, jnp.float32(0.0))

            maxv = jnp.zeros((L,), f32)
            for c in range(NCH):
                idx = src_v[pl.ds(c * L, L)]
                active = sinv_v[pl.ds(c * L, L)] > 0
                sc = jnp.where(active, plsc.load_gather(kq_v, [idx]), 0.0)
                d_v[pl.ds(c * L, L)] = sc
                maxv = jnp.maximum(maxv, sc)
            m2 = jnp.max(maxv)
            sumv = jnp.zeros((L,), f32)
            for c in range(NCH):
                e = jnp.exp(d_v[pl.ds(c * L, L)] - m2)
                d_v[pl.ds(c * L, L)] = e
                sumv = sumv + e
            ez = jnp.min(jnp.exp(zf - m2))
            denom = jnp.sum(sumv) + (MEMORY_SLOTS_FULL - M) * ez
            for c in range(NCH):
                attn_v[pl.ds(c * L, L)] = zf
            for c in range(NCH):
                idx = src_v[pl.ds(c * L, L)]
                active = sinv_v[pl.ds(c * L, L)] > 0
                plsc.store_scatter(attn_v, [idx],
                                   d_v[pl.ds(c * L, L)] / denom, mask=active)
            pltpu.sync_copy(attn_v, attn_hbm.at[b])
            nw_v[...] = jnp.full((L,), writes, f32)
            pltpu.sync_copy(nw_v, nw_hbm.at[b])

    return scan_kernel(G, n2, invn, wval, kq)


def _post_kernel(attn_ref, enc_ref, q_ref, Wo_ref, bo_ref, nw_ref,
                 logits_ref, wr_ref):
    B, T, H = enc_ref.shape
    f32 = jnp.float32
    attn = attn_ref[...]
    retrieved = jnp.concatenate(
        [lax.dot_general(attn[b:b + 1, :], enc_ref[b],
                         (((1,), (0,)), ((), ())),
                         preferred_element_type=f32)
         for b in range(B)], axis=0)
    logits_ref[...] = (_dot_t(retrieved + q_ref[...], Wo_ref[...])
                       + bo_ref[...])
    total = jnp.sum(nw_ref[...][:, 0:1])
    wr_ref[...] = (total / (B * (T - 3))).reshape(1, 1)


def kernel(enc_hidden, query_hidden, Wg, bg, Wq, bq, Wk, bk, Wo, bo):
    B, T, H = enc_hidden.shape
    f32 = jnp.float32
    pre_out = (
        jax.ShapeDtypeStruct((B * T, 1), f32),
        jax.ShapeDtypeStruct((B, T, T), f32),
        jax.ShapeDtypeStruct((B * T, 1), f32),
        jax.ShapeDtypeStruct((B * T, 1), f32),
        jax.ShapeDtypeStruct((B * T, 1), f32),
        jax.ShapeDtypeStruct((B, T), f32),
    )
    gp, G, n2f, invnf, wvalf, kq = pl.pallas_call(
        _pre_kernel, out_shape=pre_out,
    )(
        enc_hidden, enc_hidden.reshape(B * T, H), query_hidden, Wg,
        bg.reshape(1, 1), Wq, bq.reshape(1, H), Wk, bk.reshape(1, H),
    )
    attnW, nw = _sc_scan(G.reshape(B, T * T), n2f.reshape(B, T),
                         invnf.reshape(B, T), wvalf.reshape(B, T), kq)
    post_out = (
        jax.ShapeDtypeStruct((B, VOCAB_SIZE), f32),
        jax.ShapeDtypeStruct((1, 1), f32),
    )
    logits, wr = pl.pallas_call(
        _post_kernel, out_shape=post_out,
    )(attnW, enc_hidden, query_hidden, Wo, bo.reshape(1, VOCAB_SIZE), nw)
    return logits, gp.reshape(B, T), wr[0, 0]


# speedup vs baseline: 14.8179x; 1.1015x over previous
import functools

import jax
import jax.numpy as jnp
from jax import lax
from jax.experimental import pallas as pl
from jax.experimental.pallas import tpu as pltpu
from jax.experimental.pallas import tpu_sc as plsc

HIDDEN_DIM = 256
MEMORY_SLOTS_FULL = 512
MEMORY_SLOTS = 128
VOCAB_SIZE = 64
EPS = 1e-8
LANES = 16


def _dot_t(a, b):
    return jax.lax.dot_general(
        a, b, (((1,), (1,)), ((), ())), preferred_element_type=jnp.float32)


def _pre_kernel(enc_ref, enc2d_ref, q_ref, Wg_ref, bg_ref, Wq_ref, bq_ref,
                Wk_ref, bk_ref,
                gp_ref, G_ref, n2f_ref, invnf_ref, wvalf_ref, kq_ref):
    B, T, H = enc_ref.shape
    f32 = jnp.float32
    bg = bg_ref[0, 0]
    enc2d = enc2d_ref[...]
    wg8_bf = Wg_ref[...].astype(jnp.bfloat16)
    gs = (_dot_t(enc2d.astype(jnp.bfloat16), wg8_bf)[:, 0:1]
          + bg)
    gp_ref[...] = jax.nn.sigmoid(gs)
    n2f = jnp.sum(enc2d * enc2d, axis=-1, keepdims=True)
    n2f_ref[...] = n2f
    invnf_ref[...] = 1.0 / jnp.maximum(jnp.sqrt(n2f), EPS)
    wvalf_ref[...] = jnp.where(gs > 0, n2f, -1.0)
    q = _dot_t(q_ref[...], Wq_ref[...]) + bq_ref[...]
    u = lax.dot_general(q, Wk_ref[...], (((1,), (0,)), ((), ())),
                        preferred_element_type=f32)
    scale = 1.0 / (H ** 0.5)
    for b in range(B):
        eb = enc_ref[b].astype(jnp.bfloat16)
        G_ref[b] = _dot_t(eb, eb)
    kq_ref[...] = jnp.concatenate(
        [_dot_t(u[b:b + 1, :], enc_ref[b]) for b in range(B)],
        axis=0) * scale


def _sc_scan(G, n2, invn, wval, kq):
    B, T = n2.shape
    M = MEMORY_SLOTS
    L = LANES
    NCH = M // L
    f32 = jnp.float32
    i32 = jnp.int32
    mesh = plsc.VectorSubcoreMesh(core_axis_name="c", subcore_axis_name="s")

    @functools.partial(
        pl.kernel,
        mesh=mesh,
        compiler_params=pltpu.CompilerParams(needs_layout_passes=False),
        out_type=[jax.ShapeDtypeStruct((B, T), f32),
                  jax.ShapeDtypeStruct((B, L), f32)],
        scratch_types=[
            pltpu.VMEM((T * T,), f32),
            pltpu.VMEM((T,), f32),
            pltpu.VMEM((T,), f32),
            pltpu.VMEM((T,), f32),
            pltpu.VMEM((T,), f32),
            pltpu.VMEM((T,), f32),
            pltpu.VMEM((L,), f32),
        ],
    )
    def scan_kernel(G_hbm, n2_hbm, invn_hbm, wval_hbm, kq_hbm,
                    attn_hbm, nw_hbm,
                    g_v, n2_v, invn_v, wval_v, kq_v, attn_v, nw_v):
        wid = lax.axis_index("s") * 2 + lax.axis_index("c")

        @pl.when(wid < B)
        def _body():
            b = wid
            pltpu.sync_copy(G_hbm.at[b], g_v)
            pltpu.sync_copy(n2_hbm.at[b], n2_v)
            pltpu.sync_copy(invn_hbm.at[b], invn_v)
            pltpu.sync_copy(wval_hbm.at[b], wval_v)
            pltpu.sync_copy(kq_hbm.at[b], kq_v)
            zf = jnp.zeros((L,), f32)
            zi = jnp.zeros((L,), i32)
            lane_iota = lax.broadcasted_iota(i32, (L,), 0)
            BIG = jnp.int32(1 << 30)
            INF = jnp.full((L,), jnp.inf, f32)

            def step(t, carry):
                src, sinv, sn2, writes = carry
                tbase = t * T
                minv, idxv, n2v = INF, jnp.full((L,), BIG, i32), zf
                for c in range(NCH):
                    d = plsc.load_gather(g_v, [src[c] + tbase]) * sinv[c]
                    lt = d < minv
                    minv = jnp.where(lt, d, minv)
                    idxv = jnp.where(lt, lane_iota + c * L, idxv)
                    n2v = jnp.where(lt, sn2[c], n2v)
                m = jnp.min(minv)
                best = jnp.min(jnp.where(minv == m, idxv, BIG))
                n2old = jnp.min(jnp.where(idxv == best, n2v, INF))
                tx = jnp.full((L,), t, i32)
                wv = plsc.load_gather(wval_v, [tx])
                do = jnp.min(wv) > n2old
                dovec = (zi + jnp.where(do, 1, 0)) > 0
                invt = plsc.load_gather(invn_v, [tx])
                n2t = plsc.load_gather(n2_v, [tx])
                src, sinv, sn2 = list(src), list(sinv), list(sn2)
                for c in range(NCH):
                    hit = jnp.logical_and(lane_iota + c * L == best, dovec)
                    src[c] = jnp.where(hit, tx, src[c])
                    sinv[c] = jnp.where(hit, invt, sinv[c])
                    sn2[c] = jnp.where(hit, n2t, sn2[c])
                writes = writes + jnp.where(do, 1.0, 0.0)
                return tuple(src), tuple(sinv), tuple(sn2), writes

            init = (tuple(zi for _ in range(NCH)),
                    tuple(zf for _ in range(NCH)),
                    tuple(zf for _ in range(NCH)),
                    jnp.float32(0.0))
            src, sinv, sn2, writes = lax.fori_loop(0, T - 3, step, init)

            maxv = zf
            scs = []
            for c in range(NCH):
                sc = jnp.where(sinv[c] > 0,
                               plsc.load_gather(kq_v, [src[c]]), 0.0)
                scs.append(sc)
                maxv = jnp.maximum(maxv, sc)
            m2 = jnp.max(maxv)
            sumv = zf
            es = []
            for c in range(NCH):
                e = jnp.exp(scs[c] - m2)
                es.append(e)
                sumv = sumv + e
            ez = jnp.min(jnp.exp(zf - m2))
            denom = jnp.sum(sumv) + (MEMORY_SLOTS_FULL - M) * ez
            for c in range(NCH):
                attn_v[pl.ds(c * L, L)] = zf
            for c in range(NCH):
                plsc.store_scatter(attn_v, [src[c]], es[c] / denom,
                                   mask=sinv[c] > 0)
            pltpu.sync_copy(attn_v, attn_hbm.at[b])
            nw_v[...] = jnp.full((L,), writes, f32)
            pltpu.sync_copy(nw_v, nw_hbm.at[b])

    return scan_kernel(G, n2, invn, wval, kq)


def _post_kernel(attn_ref, enc_ref, q_ref, Wo_ref, bo_ref, nw_ref,
                 logits_ref, wr_ref):
    B, T, H = enc_ref.shape
    f32 = jnp.float32
    attn = attn_ref[...]
    retrieved = jnp.concatenate(
        [lax.dot_general(attn[b:b + 1, :], enc_ref[b],
                         (((1,), (0,)), ((), ())),
                         preferred_element_type=f32)
         for b in range(B)], axis=0)
    logits_ref[...] = (_dot_t(retrieved + q_ref[...], Wo_ref[...])
                       + bo_ref[...])
    total = jnp.sum(nw_ref[...][:, 0:1])
    wr_ref[...] = (total / (B * (T - 3))).reshape(1, 1)


def kernel(enc_hidden, query_hidden, Wg, bg, Wq, bq, Wk, bk, Wo, bo):
    B, T, H = enc_hidden.shape
    f32 = jnp.float32
    pre_out = (
        jax.ShapeDtypeStruct((B * T, 1), f32),
        jax.ShapeDtypeStruct((B, T, T), f32),
        jax.ShapeDtypeStruct((B * T, 1), f32),
        jax.ShapeDtypeStruct((B * T, 1), f32),
        jax.ShapeDtypeStruct((B * T, 1), f32),
        jax.ShapeDtypeStruct((B, T), f32),
    )
    gp, G, n2f, invnf, wvalf, kq = pl.pallas_call(
        _pre_kernel, out_shape=pre_out,
    )(
        enc_hidden, enc_hidden.reshape(B * T, H), query_hidden,
        jnp.broadcast_to(Wg, (8, H)),
        bg.reshape(1, 1), Wq, bq.reshape(1, H), Wk, bk.reshape(1, H),
    )
    attnW, nw = _sc_scan(G.reshape(B, T * T), n2f.reshape(B, T),
                         invnf.reshape(B, T), wvalf.reshape(B, T), kq)
    post_out = (
        jax.ShapeDtypeStruct((B, VOCAB_SIZE), f32),
        jax.ShapeDtypeStruct((1, 1), f32),
    )
    logits, wr = pl.pallas_call(
        _post_kernel, out_shape=post_out,
    )(attnW, enc_hidden, query_hidden, Wo, bo.reshape(1, VOCAB_SIZE), nw)
    return logits, gp.reshape(B, T), wr[0, 0]
